# baseline (speedup 1.0000x reference)
import functools

import jax
import jax.numpy as jnp
from jax import lax
from jax.experimental import pallas as pl
from jax.experimental.pallas import tpu as pltpu
from jax.experimental.pallas import tpu_sc as plsc

_C = 160
_CH = 80
_L = 16


def _pack_tables(scale_w, shift_w):
    su = lax.bitcast_convert_type(scale_w.astype(jnp.bfloat16), jnp.uint16)
    hu = lax.bitcast_convert_type(shift_w.astype(jnp.bfloat16), jnp.uint16)
    w = (hu.astype(jnp.uint32) << 16) | su.astype(jnp.uint32)
    return lax.bitcast_convert_type(w, jnp.int32)


def kernel(inputs, z, scale_w, shift_w):
    n, d = inputs.shape
    t = scale_w.shape[0]
    tab = _pack_tables(scale_w, shift_w)
    z32 = z.astype(jnp.int32)
    num_chunks = n // _C
    info = plsc.get_sparse_core_info()
    nw = info.num_cores * info.num_subcores
    cpw = -(-num_chunks // nw)

    @functools.partial(
        pl.kernel,
        out_type=jax.ShapeDtypeStruct((n, d), jnp.float32),
        mesh=plsc.VectorSubcoreMesh(core_axis_name="c", subcore_axis_name="s"),
        scratch_types=[
            pltpu.VMEM((t, d), jnp.int32),
            pltpu.VMEM((2, 2, _CH), jnp.int32),
            pltpu.VMEM((2, _C, d), jnp.float32),
            pltpu.VMEM((2, _C, d), jnp.float32),
            pltpu.SemaphoreType.DMA((2,)),
            pltpu.SemaphoreType.DMA((2,)),
            pltpu.SemaphoreType.DMA((2,)),
        ],
        compiler_params=pltpu.CompilerParams(needs_layout_passes=False),
    )
    def run(tab_hbm, x_hbm, z_hbm, out_hbm, tab_v, idx_v, x_v, o_v,
            sem_i, sem_x, sem_s):
        wid = lax.axis_index("s") * info.num_cores + lax.axis_index("c")
        start = wid * cpw
        count = jnp.maximum(jnp.minimum(cpw, num_chunks - start), 0)

        def row_base(tt):
            return (start + tt) * _C

        def start_idx(tt):
            for h in range(2):
                pltpu.async_copy(
                    z_hbm.at[pl.ds(row_base(tt) + h * _CH, _CH)],
                    idx_v.at[tt % 2, h],
                    sem_i.at[tt % 2],
                )

        def wait_idx(tt):
            for h in range(2):
                pltpu.make_async_copy(
                    z_hbm.at[pl.ds(row_base(tt) + h * _CH, _CH)],
                    idx_v.at[tt % 2, h],
                    sem_i.at[tt % 2],
                ).wait()

        def start_x(tt):
            for h in range(2):
                pltpu.async_copy(
                    x_hbm.at[pl.ds(row_base(tt) + h * _CH, _CH), :],
                    x_v.at[tt % 2, pl.ds(h * _CH, _CH)],
                    sem_x.at[tt % 2],
                )

        def wait_x(tt):
            for h in range(2):
                pltpu.make_async_copy(
                    x_hbm.at[pl.ds(row_base(tt) + h * _CH, _CH), :],
                    x_v.at[tt % 2, pl.ds(h * _CH, _CH)],
                    sem_x.at[tt % 2],
                ).wait()

        def start_out_half(tt, h):
            pltpu.async_copy(
                o_v.at[tt % 2, pl.ds(h * _CH, _CH)],
                out_hbm.at[pl.ds(row_base(tt) + h * _CH, _CH), :],
                sem_s.at[tt % 2],
            )

        def wait_out(tt):
            for h in range(2):
                pltpu.make_async_copy(
                    o_v.at[tt % 2, pl.ds(h * _CH, _CH)],
                    out_hbm.at[pl.ds(row_base(tt) + h * _CH, _CH), :],
                    sem_s.at[tt % 2],
                ).wait()

        start_idx(0)
        start_x(0)
        pltpu.sync_copy(tab_hbm, tab_v)

        @pl.when(count > 1)
        def _():
            start_idx(1)
            start_x(1)

        cols = [
            lax.iota(jnp.int32, _L) + jnp.full((_L,), j * _L, jnp.int32)
            for j in range(d // _L)
        ]
        shift16 = jnp.full((_L,), 16, jnp.int32)
        mask_hi = jnp.full((_L,), -65536, jnp.int32)

        def chunk(tt, carry):
            wait_idx(tt)
            wait_x(tt)

            @pl.when(tt >= 2)
            def _():
                wait_out(tt - 2)

            slot = tt % 2
            slot_vec = jnp.full((_L,), slot, jnp.int32)
            x_s = x_v.at[slot]
            o_s = o_v.at[slot]

            for h in range(2):
                h_vec = jnp.full((_L,), h, jnp.int32)

                @plsc.parallel_loop(0, _CH, step=1, unroll=8)
                def row(i):
                    zv = plsc.load_gather(
                        idx_v, [slot_vec, h_vec, jnp.full((_L,), i, jnp.int32)]
                    )
                    r = i + h * _CH
                    for j in range(d // _L):
                        w = plsc.load_gather(tab_v, [zv, cols[j]])
                        scale = plsc.bitcast(lax.shift_left(w, shift16), jnp.float32)
                        shift = plsc.bitcast(lax.bitwise_and(w, mask_hi), jnp.float32)
                        o_s[r, pl.ds(j * _L, _L)] = (
                            x_s[r, pl.ds(j * _L, _L)] * scale + shift
                        )

                start_out_half(tt, h)

            @pl.when(tt + 2 < count)
            def _():
                start_idx(tt + 2)
                start_x(tt + 2)

            return carry

        lax.fori_loop(0, count, chunk, 0)

        @pl.when(count >= 2)
        def _():
            wait_out(count - 2)

        wait_out(count - 1)

    return run(tab, inputs, z32)


# speedup vs baseline: 1.0475x; 1.0475x over previous
import functools

import jax
import jax.numpy as jnp
from jax import lax
from jax.experimental import pallas as pl
from jax.experimental.pallas import tpu as pltpu
from jax.experimental.pallas import tpu_sc as plsc

_C = 160
_CH = 80
_L = 16


def _pack_tables(scale_w, shift_w):
    su = lax.bitcast_convert_type(scale_w.astype(jnp.bfloat16), jnp.uint16)
    hu = lax.bitcast_convert_type(shift_w.astype(jnp.bfloat16), jnp.uint16)
    w = (hu.astype(jnp.uint32) << 16) | su.astype(jnp.uint32)
    return lax.bitcast_convert_type(w, jnp.int32)


def kernel(inputs, z, scale_w, shift_w):
    n, d = inputs.shape
    t = scale_w.shape[0]
    tab = _pack_tables(scale_w, shift_w)
    z32 = z.astype(jnp.int32)
    num_chunks = n // _C
    info = plsc.get_sparse_core_info()
    nw = info.num_cores * info.num_subcores
    cpw = -(-num_chunks // nw)

    @functools.partial(
        pl.kernel,
        out_type=jax.ShapeDtypeStruct((n, d), jnp.float32),
        mesh=plsc.VectorSubcoreMesh(core_axis_name="c", subcore_axis_name="s"),
        scratch_types=[
            pltpu.VMEM((t, d), jnp.int32),
            pltpu.VMEM((2, 2, _CH), jnp.int32),
            pltpu.VMEM((2, _C, d), jnp.float32),
            pltpu.VMEM((2, _C, d), jnp.float32),
            pltpu.SemaphoreType.DMA((2,)),
            pltpu.SemaphoreType.DMA((2,)),
            pltpu.SemaphoreType.DMA((2,)),
        ],
        compiler_params=pltpu.CompilerParams(needs_layout_passes=False),
    )
    def run(tab_hbm, x_hbm, z_hbm, out_hbm, tab_v, idx_v, x_v, o_v,
            sem_i, sem_x, sem_s):
        wid = lax.axis_index("s") * info.num_cores + lax.axis_index("c")
        start = wid * cpw
        count = jnp.maximum(jnp.minimum(cpw, num_chunks - start), 0)

        def row_base(tt):
            return (start + tt) * _C

        def start_idx(tt):
            for h in range(2):
                pltpu.async_copy(
                    z_hbm.at[pl.ds(row_base(tt) + h * _CH, _CH)],
                    idx_v.at[tt % 2, h],
                    sem_i.at[tt % 2],
                )

        def wait_idx(tt):
            for h in range(2):
                pltpu.make_async_copy(
                    z_hbm.at[pl.ds(row_base(tt) + h * _CH, _CH)],
                    idx_v.at[tt % 2, h],
                    sem_i.at[tt % 2],
                ).wait()

        def start_x(tt):
            for h in range(2):
                pltpu.async_copy(
                    x_hbm.at[pl.ds(row_base(tt) + h * _CH, _CH), :],
                    x_v.at[tt % 2, pl.ds(h * _CH, _CH)],
                    sem_x.at[tt % 2],
                )

        def wait_x_half(tt, h):
            pltpu.make_async_copy(
                x_hbm.at[pl.ds(row_base(tt) + h * _CH, _CH), :],
                x_v.at[tt % 2, pl.ds(h * _CH, _CH)],
                sem_x.at[tt % 2],
            ).wait()

        def start_out_half(tt, h):
            pltpu.async_copy(
                o_v.at[tt % 2, pl.ds(h * _CH, _CH)],
                out_hbm.at[pl.ds(row_base(tt) + h * _CH, _CH), :],
                sem_s.at[tt % 2],
            )

        def wait_out(tt):
            for h in range(2):
                pltpu.make_async_copy(
                    o_v.at[tt % 2, pl.ds(h * _CH, _CH)],
                    out_hbm.at[pl.ds(row_base(tt) + h * _CH, _CH), :],
                    sem_s.at[tt % 2],
                ).wait()

        start_idx(0)
        start_x(0)
        pltpu.sync_copy(tab_hbm, tab_v)

        @pl.when(count > 1)
        def _():
            start_idx(1)
            start_x(1)

        cols = [
            lax.iota(jnp.int32, _L) + jnp.full((_L,), j * _L, jnp.int32)
            for j in range(d // _L)
        ]
        shift16 = jnp.full((_L,), 16, jnp.int32)
        mask_hi = jnp.full((_L,), -65536, jnp.int32)

        def chunk(tt, carry):
            @pl.when(tt >= 2)
            def _():
                wait_out(tt - 2)

            wait_idx(tt)

            slot = tt % 2
            slot_vec = jnp.full((_L,), slot, jnp.int32)
            x_s = x_v.at[slot]
            o_s = o_v.at[slot]

            for h in range(2):
                wait_x_half(tt, h)
                h_vec = jnp.full((_L,), h, jnp.int32)

                @plsc.parallel_loop(0, _CH, step=1, unroll=4)
                def row(i):
                    zv = plsc.load_gather(
                        idx_v, [slot_vec, h_vec, jnp.full((_L,), i, jnp.int32)]
                    )
                    r = i + h * _CH
                    for j in range(d // _L):
                        w = plsc.load_gather(tab_v, [zv, cols[j]])
                        scale = plsc.bitcast(lax.shift_left(w, shift16), jnp.float32)
                        shift = plsc.bitcast(lax.bitwise_and(w, mask_hi), jnp.float32)
                        o_s[r, pl.ds(j * _L, _L)] = (
                            x_s[r, pl.ds(j * _L, _L)] * scale + shift
                        )

                start_out_half(tt, h)

            @pl.when(tt + 2 < count)
            def _():
                start_idx(tt + 2)
                start_x(tt + 2)

            return carry

        lax.fori_loop(0, count, chunk, 0)

        @pl.when(count >= 2)
        def _():
            wait_out(count - 2)

        wait_out(count - 1)

    return run(tab, inputs, z32)


# speedup vs baseline: 1.0711x; 1.0226x over previous
import functools

import jax
import jax.numpy as jnp
from jax import lax
from jax.experimental import pallas as pl
from jax.experimental.pallas import tpu as pltpu
from jax.experimental.pallas import tpu_sc as plsc

_C = 160
_CH = 80
_L = 16


def _pack_tables(scale_w, shift_w):
    su = lax.bitcast_convert_type(scale_w.astype(jnp.bfloat16), jnp.uint16)
    hu = lax.bitcast_convert_type(shift_w.astype(jnp.bfloat16), jnp.uint16)
    w = (hu.astype(jnp.uint32) << 16) | su.astype(jnp.uint32)
    return lax.bitcast_convert_type(w, jnp.int32)


def kernel(inputs, z, scale_w, shift_w):
    n, d = inputs.shape
    t = scale_w.shape[0]
    tab = _pack_tables(scale_w, shift_w)
    z32 = z.astype(jnp.int32)
    num_chunks = n // _C
    info = plsc.get_sparse_core_info()
    nw = info.num_cores * info.num_subcores
    cpw = -(-num_chunks // nw)

    @functools.partial(
        pl.kernel,
        out_type=jax.ShapeDtypeStruct((n, d), jnp.float32),
        mesh=plsc.VectorSubcoreMesh(core_axis_name="c", subcore_axis_name="s"),
        scratch_types=[
            pltpu.VMEM((t, d), jnp.int32),
            pltpu.VMEM((2, 2, _CH), jnp.int32),
            pltpu.VMEM((3, _C, d), jnp.float32),
            pltpu.VMEM((2, _C, d), jnp.float32),
            pltpu.SemaphoreType.DMA((2,)),
            pltpu.SemaphoreType.DMA((3,)),
            pltpu.SemaphoreType.DMA((2,)),
        ],
        compiler_params=pltpu.CompilerParams(needs_layout_passes=False),
    )
    def run(tab_hbm, x_hbm, z_hbm, out_hbm, tab_v, idx_v, x_v, o_v,
            sem_i, sem_x, sem_s):
        wid = lax.axis_index("s") * info.num_cores + lax.axis_index("c")
        start = wid * cpw
        count = jnp.maximum(jnp.minimum(cpw, num_chunks - start), 0)

        def row_base(tt):
            return (start + tt) * _C

        def start_idx(tt):
            for h in range(2):
                pltpu.async_copy(
                    z_hbm.at[pl.ds(row_base(tt) + h * _CH, _CH)],
                    idx_v.at[tt % 2, h],
                    sem_i.at[tt % 2],
                )

        def wait_idx(tt):
            for h in range(2):
                pltpu.make_async_copy(
                    z_hbm.at[pl.ds(row_base(tt) + h * _CH, _CH)],
                    idx_v.at[tt % 2, h],
                    sem_i.at[tt % 2],
                ).wait()

        def start_x(tt):
            for h in range(2):
                pltpu.async_copy(
                    x_hbm.at[pl.ds(row_base(tt) + h * _CH, _CH), :],
                    x_v.at[tt % 3, pl.ds(h * _CH, _CH)],
                    sem_x.at[tt % 3],
                )

        def wait_x_half(tt, h):
            pltpu.make_async_copy(
                x_hbm.at[pl.ds(row_base(tt) + h * _CH, _CH), :],
                x_v.at[tt % 3, pl.ds(h * _CH, _CH)],
                sem_x.at[tt % 3],
            ).wait()

        def start_out_half(tt, h):
            pltpu.async_copy(
                o_v.at[tt % 2, pl.ds(h * _CH, _CH)],
                out_hbm.at[pl.ds(row_base(tt) + h * _CH, _CH), :],
                sem_s.at[tt % 2],
            )

        def wait_out(tt):
            for h in range(2):
                pltpu.make_async_copy(
                    o_v.at[tt % 2, pl.ds(h * _CH, _CH)],
                    out_hbm.at[pl.ds(row_base(tt) + h * _CH, _CH), :],
                    sem_s.at[tt % 2],
                ).wait()

        start_idx(0)
        start_x(0)
        pltpu.sync_copy(tab_hbm, tab_v)

        @pl.when(count > 1)
        def _():
            start_idx(1)
            start_x(1)

        cols = [
            lax.iota(jnp.int32, _L) + jnp.full((_L,), j * _L, jnp.int32)
            for j in range(d // _L)
        ]
        shift16 = jnp.full((_L,), 16, jnp.int32)
        mask_hi = jnp.full((_L,), -65536, jnp.int32)

        def chunk(tt, carry):
            @pl.when(tt + 2 < count)
            def _():
                start_x(tt + 2)

            @pl.when(tt >= 2)
            def _():
                wait_out(tt - 2)

            wait_idx(tt)

            slot = tt % 2
            slot_vec = jnp.full((_L,), slot, jnp.int32)
            x_s = x_v.at[tt % 3]
            o_s = o_v.at[slot]

            for h in range(2):
                wait_x_half(tt, h)
                h_vec = jnp.full((_L,), h, jnp.int32)

                @plsc.parallel_loop(0, _CH, step=1, unroll=4)
                def row(i):
                    zv = plsc.load_gather(
                        idx_v, [slot_vec, h_vec, jnp.full((_L,), i, jnp.int32)]
                    )
                    r = i + h * _CH
                    for j in range(d // _L):
                        w = plsc.load_gather(tab_v, [zv, cols[j]])
                        scale = plsc.bitcast(lax.shift_left(w, shift16), jnp.float32)
                        shift = plsc.bitcast(lax.bitwise_and(w, mask_hi), jnp.float32)
                        o_s[r, pl.ds(j * _L, _L)] = (
                            x_s[r, pl.ds(j * _L, _L)] * scale + shift
                        )

                start_out_half(tt, h)

            @pl.when(tt + 2 < count)
            def _():
                start_idx(tt + 2)

            return carry

        lax.fori_loop(0, count, chunk, 0)

        @pl.when(count >= 2)
        def _():
            wait_out(count - 2)

        wait_out(count - 1)

    return run(tab, inputs, z32)
